# baseline (speedup 1.0000x reference)
import functools

import jax
import jax.numpy as jnp
from jax import lax
from jax.experimental import pallas as pl
from jax.experimental.pallas import tpu as pltpu
from jax.experimental.pallas import tpu_sc as plsc

_NW = 32
_L = 16


def _rsqrt(x):
    bits = plsc.bitcast(x, jnp.int32)
    y = plsc.bitcast(jnp.full((_L,), 0x5F3759DF, jnp.int32) - (bits >> 1),
                     jnp.float32)
    half = x * 0.5
    for _ in range(3):
        y = y * (1.5 - half * y * y)
    return y


@functools.lru_cache(maxsize=None)
def _make_sc_kernel(N, E, G, C):
    EW = E // _NW
    R = C // _L
    NCH = EW // C
    mesh = plsc.VectorSubcoreMesh(core_axis_name="c", subcore_axis_name="s")

    f32 = jnp.float32
    i32 = jnp.int32

    buf_shapes = [
        ((C,), i32),
        ((C,), i32),
        ((3 * C,), f32),
        ((C,), f32),
        ((C,), f32),
        ((C,), f32),
        ((C,), f32),
        ((C,), f32),
        ((C,), f32),
        ((C,), f32),
    ]
    scratch = [pltpu.VMEM((G * 9,), f32), pltpu.VMEM((N,), i32)]
    for _ in range(2):
        scratch += [pltpu.VMEM(s, d) for s, d in buf_shapes]
        scratch += [pltpu.SemaphoreType.DMA] * 4

    @functools.partial(
        pl.kernel,
        mesh=mesh,
        out_type=jax.ShapeDtypeStruct((E,), f32),
        compiler_params=pltpu.CompilerParams(needs_layout_passes=False),
        scratch_types=scratch,
    )
    def k(px_hbm, py_hbm, pz_hbm, bat_hbm, lat_hbm,
          src_hbm, dst_hbm, shf_hbm, out_hbm,
          lat_v, bat_v, *bufs):
        nb = len(buf_shapes) + 4
        B = []
        for p in range(2):
            (src_v, dst_v, sh_v, sx_v, sy_v, sz_v, dx_v, dy_v, dz_v,
             out_v, s_idx, s_sh, s_g, s_out) = bufs[p * nb:(p + 1) * nb]
            B.append(dict(src=src_v, dst=dst_v, sh=sh_v,
                          g=(sx_v, sy_v, sz_v, dx_v, dy_v, dz_v),
                          out=out_v, s_idx=s_idx, s_sh=s_sh, s_g=s_g,
                          s_out=s_out))
        wid = lax.axis_index("s") * 2 + lax.axis_index("c")
        pltpu.sync_copy(lat_hbm, lat_v)
        pltpu.sync_copy(bat_hbm, bat_v)
        lanes = lax.iota(i32, _L)
        tabs = (px_hbm, py_hbm, pz_hbm, px_hbm, py_hbm, pz_hbm)

        def issue_idx(i, b):
            e0 = wid * EW + i * C
            pltpu.async_copy(src_hbm.at[pl.ds(e0, C)], b["src"], b["s_idx"])
            pltpu.async_copy(dst_hbm.at[pl.ds(e0, C)], b["dst"], b["s_idx"])
            pltpu.async_copy(shf_hbm.at[pl.ds(e0 * 3, 3 * C)], b["sh"],
                             b["s_sh"])

        def wait_idx(b):
            pltpu.make_async_copy(src_hbm.at[pl.ds(0, C)], b["src"],
                                  b["s_idx"]).wait()
            pltpu.make_async_copy(src_hbm.at[pl.ds(0, C)], b["dst"],
                                  b["s_idx"]).wait()

        def issue_g(b):
            for t in range(6):
                idx = b["src"] if t < 3 else b["dst"]
                pltpu.async_copy(tabs[t].at[idx], b["g"][t], b["s_g"])

        def wait_g(b):
            for t in range(6):
                pltpu.make_async_copy(px_hbm.at[pl.ds(0, C)], b["g"][t],
                                      b["s_g"]).wait()
            pltpu.make_async_copy(shf_hbm.at[pl.ds(0, 3 * C)], b["sh"],
                                  b["s_sh"]).wait()

        def wait_out(b):
            pltpu.make_async_copy(b["out"], out_hbm.at[pl.ds(0, C)],
                                  b["s_out"]).wait()

        issue_idx(0, B[0])
        issue_idx(1, B[1])
        pltpu.async_copy(px_hbm.at[pl.ds(0, C)], B[0]["out"], B[0]["s_out"])
        pltpu.async_copy(px_hbm.at[pl.ds(0, C)], B[1]["out"], B[1]["s_out"])
        wait_idx(B[0])
        issue_g(B[0])

        def compute(b, i):
            sx_v, sy_v, sz_v, dx_v, dy_v, dz_v = b["g"]
            src_v, sh_v, out_v = b["src"], b["sh"], b["out"]

            def grp(g, carry2):
                q = pl.ds(g * _L, _L)
                erow = g * _L + lanes
                e3 = erow * 3
                b9 = plsc.load_gather(bat_v, [src_v[q]]) * 9
                lat = [plsc.load_gather(lat_v, [b9 + kk]) for kk in range(9)]
                sh0 = plsc.load_gather(sh_v, [e3])
                sh1 = plsc.load_gather(sh_v, [e3 + 1])
                sh2 = plsc.load_gather(sh_v, [e3 + 2])
                vx = dx_v[q] - sx_v[q] + sh0 * lat[0] + sh1 * lat[3] \
                    + sh2 * lat[6]
                vy = dy_v[q] - sy_v[q] + sh0 * lat[1] + sh1 * lat[4] \
                    + sh2 * lat[7]
                vz = dz_v[q] - sz_v[q] + sh0 * lat[2] + sh1 * lat[5] \
                    + sh2 * lat[8]
                n2 = jnp.maximum(vx * vx + vy * vy + vz * vz, 1e-30)
                out_v[q] = n2 * _rsqrt(n2)
                return carry2

            lax.fori_loop(0, R, grp, 0)
            e0 = wid * EW + i * C
            pltpu.async_copy(out_v, out_hbm.at[pl.ds(e0, C)], b["s_out"])

        def step(t, carry):
            for p in range(2):
                i = 2 * t + p
                bp, bq = B[p], B[1 - p]

                @pl.when(i + 1 < NCH)
                def _():
                    wait_idx(bq)
                    issue_g(bq)

                wait_g(bp)
                wait_out(bp)
                compute(bp, i)

                @pl.when(i + 2 < NCH)
                def _():
                    issue_idx(i + 2, bp)

            return carry

        lax.fori_loop(0, NCH // 2, step, 0)
        wait_out(B[0])
        wait_out(B[1])

    return k


def kernel(pos, edge_index, edge_shift, lattice, batch):
    N = pos.shape[0]
    E = edge_index.shape[1]
    G = lattice.shape[0]
    latf = lattice.reshape(G * 9)
    shf = edge_shift.reshape(3 * E)
    ew = E // _NW
    c = min(800, ew)
    c -= c % _L
    while c > _L and (ew % c or (ew // c) % 2):
        c -= _L
    k = _make_sc_kernel(N, E, G, c)
    return k(pos[:, 0], pos[:, 1], pos[:, 2], batch, latf,
             edge_index[0], edge_index[1], shf)


# speedup vs baseline: 1.0008x; 1.0008x over previous
import functools

import jax
import jax.numpy as jnp
from jax import lax
from jax.experimental import pallas as pl
from jax.experimental.pallas import tpu as pltpu
from jax.experimental.pallas import tpu_sc as plsc

_NW = 32
_L = 16


def _rsqrt(x):
    bits = plsc.bitcast(x, jnp.int32)
    y = plsc.bitcast(jnp.full((_L,), 0x5F3759DF, jnp.int32) - (bits >> 1),
                     jnp.float32)
    half = x * 0.5
    for _ in range(3):
        y = y * (1.5 - half * y * y)
    return y


@functools.lru_cache(maxsize=None)
def _make_sc_kernel(N, E, G, C):
    EW = E // _NW
    R = C // _L
    NCH = EW // C
    mesh = plsc.VectorSubcoreMesh(core_axis_name="c", subcore_axis_name="s")

    f32 = jnp.float32
    i32 = jnp.int32

    buf_shapes = [
        ((C,), i32),
        ((C,), i32),
        ((3 * C,), f32),
        ((C,), f32),
        ((C,), f32),
        ((C,), f32),
        ((C,), f32),
        ((C,), f32),
        ((C,), f32),
        ((C,), f32),
    ]
    scratch = [pltpu.VMEM((G * 9,), f32), pltpu.VMEM((N,), i32)]
    for _ in range(2):
        scratch += [pltpu.VMEM(s, d) for s, d in buf_shapes]
        scratch += [pltpu.SemaphoreType.DMA] * 4

    @functools.partial(
        pl.kernel,
        mesh=mesh,
        out_type=jax.ShapeDtypeStruct((E,), f32),
        compiler_params=pltpu.CompilerParams(needs_layout_passes=False),
        scratch_types=scratch,
    )
    def k(px_hbm, py_hbm, pz_hbm, bat_hbm, lat_hbm,
          src_hbm, dst_hbm, shf_hbm, out_hbm,
          lat_v, bat_v, *bufs):
        nb = len(buf_shapes) + 4
        B = []
        for p in range(2):
            (src_v, dst_v, sh_v, sx_v, sy_v, sz_v, dx_v, dy_v, dz_v,
             out_v, s_idx, s_sh, s_g, s_out) = bufs[p * nb:(p + 1) * nb]
            B.append(dict(src=src_v, dst=dst_v, sh=sh_v,
                          g=(sx_v, sy_v, sz_v, dx_v, dy_v, dz_v),
                          out=out_v, s_idx=s_idx, s_sh=s_sh, s_g=s_g,
                          s_out=s_out))
        wid = lax.axis_index("s") * 2 + lax.axis_index("c")
        pltpu.sync_copy(lat_hbm, lat_v)
        pltpu.sync_copy(bat_hbm, bat_v)
        lanes = lax.iota(i32, _L)
        tabs = (px_hbm, py_hbm, pz_hbm, px_hbm, py_hbm, pz_hbm)

        def issue_idx(i, b):
            e0 = wid * EW + i * C
            pltpu.async_copy(src_hbm.at[pl.ds(e0, C)], b["src"], b["s_idx"])
            pltpu.async_copy(dst_hbm.at[pl.ds(e0, C)], b["dst"], b["s_idx"])
            pltpu.async_copy(shf_hbm.at[pl.ds(e0 * 3, 3 * C)], b["sh"],
                             b["s_sh"])

        def wait_idx(b):
            pltpu.make_async_copy(src_hbm.at[pl.ds(0, C)], b["src"],
                                  b["s_idx"]).wait()
            pltpu.make_async_copy(src_hbm.at[pl.ds(0, C)], b["dst"],
                                  b["s_idx"]).wait()

        def issue_g(b):
            for t in range(6):
                idx = b["src"] if t < 3 else b["dst"]
                pltpu.async_copy(tabs[t].at[idx], b["g"][t], b["s_g"])

        def wait_g(b):
            for t in range(6):
                pltpu.make_async_copy(px_hbm.at[pl.ds(0, C)], b["g"][t],
                                      b["s_g"]).wait()
            pltpu.make_async_copy(shf_hbm.at[pl.ds(0, 3 * C)], b["sh"],
                                  b["s_sh"]).wait()

        def wait_out(b):
            pltpu.make_async_copy(b["out"], out_hbm.at[pl.ds(0, C)],
                                  b["s_out"]).wait()

        issue_idx(0, B[0])
        issue_idx(1, B[1])
        pltpu.async_copy(px_hbm.at[pl.ds(0, C)], B[0]["out"], B[0]["s_out"])
        pltpu.async_copy(px_hbm.at[pl.ds(0, C)], B[1]["out"], B[1]["s_out"])
        wait_idx(B[0])
        issue_g(B[0])

        def compute(b, i):
            sx_v, sy_v, sz_v, dx_v, dy_v, dz_v = b["g"]
            src_v, sh_v, out_v = b["src"], b["sh"], b["out"]

            def grp(g, carry2):
                q = pl.ds(g * _L, _L)
                erow = g * _L + lanes
                e3 = erow * 3
                b9 = plsc.load_gather(bat_v, [src_v[q]]) * 9
                lat = [plsc.load_gather(lat_v, [b9 + kk]) for kk in range(9)]
                sh0 = plsc.load_gather(sh_v, [e3])
                sh1 = plsc.load_gather(sh_v, [e3 + 1])
                sh2 = plsc.load_gather(sh_v, [e3 + 2])
                vx = dx_v[q] - sx_v[q] + sh0 * lat[0] + sh1 * lat[3] \
                    + sh2 * lat[6]
                vy = dy_v[q] - sy_v[q] + sh0 * lat[1] + sh1 * lat[4] \
                    + sh2 * lat[7]
                vz = dz_v[q] - sz_v[q] + sh0 * lat[2] + sh1 * lat[5] \
                    + sh2 * lat[8]
                n2 = jnp.maximum(vx * vx + vy * vy + vz * vz, 1e-30)
                out_v[q] = n2 * _rsqrt(n2)
                return carry2

            lax.fori_loop(0, R, grp, 0)
            e0 = wid * EW + i * C
            pltpu.async_copy(out_v, out_hbm.at[pl.ds(e0, C)], b["s_out"])

        def step(t, carry):
            for p in range(2):
                i = 2 * t + p
                bp, bq = B[p], B[1 - p]
                wait_idx(bq)
                issue_g(bq)
                wait_g(bp)
                wait_out(bp)
                compute(bp, i)
                issue_idx(i + 2, bp)
            return carry

        lax.fori_loop(0, NCH // 2 - 1, step, 0)
        wait_idx(B[1])
        issue_g(B[1])
        wait_g(B[0])
        wait_out(B[0])
        compute(B[0], NCH - 2)
        wait_g(B[1])
        wait_out(B[1])
        compute(B[1], NCH - 1)
        wait_out(B[0])
        wait_out(B[1])

    return k


def kernel(pos, edge_index, edge_shift, lattice, batch):
    N = pos.shape[0]
    E = edge_index.shape[1]
    G = lattice.shape[0]
    latf = lattice.reshape(G * 9)
    shf = edge_shift.reshape(3 * E)
    ew = E // _NW
    c = min(800, ew)
    c -= c % _L
    while c > _L and (ew % c or (ew // c) % 2):
        c -= _L
    k = _make_sc_kernel(N, E, G, c)
    return k(pos[:, 0], pos[:, 1], pos[:, 2], batch, latf,
             edge_index[0], edge_index[1], shf)


# speedup vs baseline: 5.4486x; 5.4445x over previous
import functools

import jax
import jax.numpy as jnp
from jax import lax
from jax.experimental import pallas as pl
from jax.experimental.pallas import tpu as pltpu
from jax.experimental.pallas import tpu_sc as plsc

_NW = 32
_L = 16


def _rsqrt(x):
    bits = plsc.bitcast(x, jnp.int32)
    y = plsc.bitcast(jnp.full((_L,), 0x5F3759DF, jnp.int32) - (bits >> 1),
                     jnp.float32)
    half = x * 0.5
    for _ in range(3):
        y = y * (1.5 - half * y * y)
    return y


@functools.lru_cache(maxsize=None)
def _make_sc_kernel(N, E, G, C):
    EW = E // _NW
    R = C // _L
    NCH = EW // C
    mesh = plsc.VectorSubcoreMesh(core_axis_name="c", subcore_axis_name="s")

    f32 = jnp.float32
    i32 = jnp.int32

    buf_shapes = [
        ((C,), i32),
        ((C,), i32),
        ((C,), f32),
        ((C,), f32),
        ((C,), f32),
        ((C,), f32),
        ((C,), f32),
        ((C,), f32),
        ((C,), f32),
        ((C,), f32),
        ((C,), f32),
        ((C,), f32),
    ]
    scratch = [pltpu.VMEM((G * 9,), f32), pltpu.VMEM((N,), i32)]
    for _ in range(2):
        scratch += [pltpu.VMEM(s, d) for s, d in buf_shapes]
        scratch += [pltpu.SemaphoreType.DMA] * 4

    @functools.partial(
        pl.kernel,
        mesh=mesh,
        out_type=jax.ShapeDtypeStruct((E,), f32),
        compiler_params=pltpu.CompilerParams(needs_layout_passes=False),
        scratch_types=scratch,
    )
    def k(px_hbm, py_hbm, pz_hbm, bat_hbm, lat_hbm,
          src_hbm, dst_hbm, s0_hbm, s1_hbm, s2_hbm, out_hbm,
          lat_v, bat_v, *bufs):
        nb = len(buf_shapes) + 4
        B = []
        for p in range(2):
            (src_v, dst_v, h0_v, h1_v, h2_v, sx_v, sy_v, sz_v,
             dx_v, dy_v, dz_v,
             out_v, s_idx, s_sh, s_g, s_out) = bufs[p * nb:(p + 1) * nb]
            B.append(dict(src=src_v, dst=dst_v, sh=(h0_v, h1_v, h2_v),
                          g=(sx_v, sy_v, sz_v, dx_v, dy_v, dz_v),
                          out=out_v, s_idx=s_idx, s_sh=s_sh, s_g=s_g,
                          s_out=s_out))
        wid = lax.axis_index("s") * 2 + lax.axis_index("c")
        pltpu.sync_copy(lat_hbm, lat_v)
        pltpu.sync_copy(bat_hbm, bat_v)
        lanes = lax.iota(i32, _L)
        tabs = (px_hbm, py_hbm, pz_hbm, px_hbm, py_hbm, pz_hbm)

        def issue_idx(i, b):
            e0 = wid * EW + i * C
            pltpu.async_copy(src_hbm.at[pl.ds(e0, C)], b["src"], b["s_idx"])
            pltpu.async_copy(dst_hbm.at[pl.ds(e0, C)], b["dst"], b["s_idx"])
            for t, hh in enumerate((s0_hbm, s1_hbm, s2_hbm)):
                pltpu.async_copy(hh.at[pl.ds(e0, C)], b["sh"][t], b["s_sh"])

        def wait_idx(b):
            pltpu.make_async_copy(src_hbm.at[pl.ds(0, C)], b["src"],
                                  b["s_idx"]).wait()
            pltpu.make_async_copy(src_hbm.at[pl.ds(0, C)], b["dst"],
                                  b["s_idx"]).wait()

        def issue_g(b):
            for t in range(6):
                idx = b["src"] if t < 3 else b["dst"]
                pltpu.async_copy(tabs[t].at[idx], b["g"][t], b["s_g"])

        def wait_g(b):
            for t in range(6):
                pltpu.make_async_copy(px_hbm.at[pl.ds(0, C)], b["g"][t],
                                      b["s_g"]).wait()
            for t in range(3):
                pltpu.make_async_copy(px_hbm.at[pl.ds(0, C)], b["sh"][t],
                                      b["s_sh"]).wait()

        def wait_out(b):
            pltpu.make_async_copy(b["out"], out_hbm.at[pl.ds(0, C)],
                                  b["s_out"]).wait()

        issue_idx(0, B[0])
        issue_idx(1, B[1])
        pltpu.async_copy(px_hbm.at[pl.ds(0, C)], B[0]["out"], B[0]["s_out"])
        pltpu.async_copy(px_hbm.at[pl.ds(0, C)], B[1]["out"], B[1]["s_out"])
        wait_idx(B[0])
        issue_g(B[0])

        def compute(b, i):
            sx_v, sy_v, sz_v, dx_v, dy_v, dz_v = b["g"]
            src_v, out_v = b["src"], b["out"]
            h0_v, h1_v, h2_v = b["sh"]

            def grp(g, carry2):
                q = pl.ds(g * _L, _L)
                b9 = plsc.load_gather(bat_v, [src_v[q]]) * 9
                lat = [plsc.load_gather(lat_v, [b9 + kk]) for kk in range(9)]
                sh0 = h0_v[q]
                sh1 = h1_v[q]
                sh2 = h2_v[q]
                vx = dx_v[q] - sx_v[q] + sh0 * lat[0] + sh1 * lat[3] \
                    + sh2 * lat[6]
                vy = dy_v[q] - sy_v[q] + sh0 * lat[1] + sh1 * lat[4] \
                    + sh2 * lat[7]
                vz = dz_v[q] - sz_v[q] + sh0 * lat[2] + sh1 * lat[5] \
                    + sh2 * lat[8]
                n2 = jnp.maximum(vx * vx + vy * vy + vz * vz, 1e-30)
                out_v[q] = n2 * _rsqrt(n2)
                return carry2

            lax.fori_loop(0, R, grp, 0)
            e0 = wid * EW + i * C
            pltpu.async_copy(out_v, out_hbm.at[pl.ds(e0, C)], b["s_out"])

        def step(t, carry):
            for p in range(2):
                i = 2 * t + p
                bp, bq = B[p], B[1 - p]
                wait_idx(bq)
                issue_g(bq)
                wait_g(bp)
                wait_out(bp)
                compute(bp, i)
                issue_idx(i + 2, bp)
            return carry

        lax.fori_loop(0, NCH // 2 - 1, step, 0)
        wait_idx(B[1])
        issue_g(B[1])
        wait_g(B[0])
        wait_out(B[0])
        compute(B[0], NCH - 2)
        wait_g(B[1])
        wait_out(B[1])
        compute(B[1], NCH - 1)
        wait_out(B[0])
        wait_out(B[1])

    return k


def kernel(pos, edge_index, edge_shift, lattice, batch):
    N = pos.shape[0]
    E = edge_index.shape[1]
    G = lattice.shape[0]
    latf = lattice.reshape(G * 9)
    ew = E // _NW
    c = min(800, ew)
    c -= c % _L
    while c > _L and (ew % c or (ew // c) % 2):
        c -= _L
    k = _make_sc_kernel(N, E, G, c)
    return k(pos[:, 0], pos[:, 1], pos[:, 2], batch, latf,
             edge_index[0], edge_index[1],
             edge_shift[:, 0], edge_shift[:, 1], edge_shift[:, 2])


# speedup vs baseline: 7.3910x; 1.3565x over previous
import functools

import jax
import jax.numpy as jnp
from jax import lax
from jax.experimental import pallas as pl
from jax.experimental.pallas import tpu as pltpu
from jax.experimental.pallas import tpu_sc as plsc

_NW = 32
_L = 16


def _rsqrt(x):
    bits = plsc.bitcast(x, jnp.int32)
    y = plsc.bitcast(jnp.full((_L,), 0x5F3759DF, jnp.int32) - (bits >> 1),
                     jnp.float32)
    half = x * 0.5
    for _ in range(3):
        y = y * (1.5 - half * y * y)
    return y


def _hi(w):
    return plsc.bitcast(w & jnp.int32(-65536), jnp.float32)


def _lo(w):
    return plsc.bitcast(w << 16, jnp.float32)


@functools.lru_cache(maxsize=None)
def _make_sc_kernel(N, E, G, C):
    EW = E // _NW
    R = C // _L
    NCH = EW // C
    mesh = plsc.VectorSubcoreMesh(core_axis_name="c", subcore_axis_name="s")

    f32 = jnp.float32
    i32 = jnp.int32

    buf_shapes = [
        ((C,), i32),
        ((C,), i32),
        ((C,), f32),
        ((C,), f32),
        ((C,), f32),
        ((C,), i32),
        ((C,), i32),
        ((C,), i32),
        ((C,), i32),
        ((C,), f32),
    ]
    scratch = [pltpu.VMEM((G * 9,), f32)]
    for _ in range(2):
        scratch += [pltpu.VMEM(s, d) for s, d in buf_shapes]
        scratch += [pltpu.SemaphoreType.DMA] * 4

    @functools.partial(
        pl.kernel,
        mesh=mesh,
        out_type=jax.ShapeDtypeStruct((E,), f32),
        compiler_params=pltpu.CompilerParams(needs_layout_passes=False),
        scratch_types=scratch,
    )
    def k(xyp_hbm, zbp_hbm, lat_hbm,
          src_hbm, dst_hbm, s0_hbm, s1_hbm, s2_hbm, out_hbm,
          lat_v, *bufs):
        nb = len(buf_shapes) + 4
        B = []
        for p in range(2):
            (src_v, dst_v, h0_v, h1_v, h2_v, xys_v, zbs_v, xyd_v, zbd_v,
             out_v, s_idx, s_sh, s_g, s_out) = bufs[p * nb:(p + 1) * nb]
            B.append(dict(src=src_v, dst=dst_v, sh=(h0_v, h1_v, h2_v),
                          g=(xys_v, zbs_v, xyd_v, zbd_v),
                          out=out_v, s_idx=s_idx, s_sh=s_sh, s_g=s_g,
                          s_out=s_out))
        wid = lax.axis_index("s") * 2 + lax.axis_index("c")
        pltpu.sync_copy(lat_hbm, lat_v)

        def issue_idx(i, b):
            e0 = wid * EW + i * C
            pltpu.async_copy(src_hbm.at[pl.ds(e0, C)], b["src"], b["s_idx"])
            pltpu.async_copy(dst_hbm.at[pl.ds(e0, C)], b["dst"], b["s_idx"])
            for t, hh in enumerate((s0_hbm, s1_hbm, s2_hbm)):
                pltpu.async_copy(hh.at[pl.ds(e0, C)], b["sh"][t], b["s_sh"])

        def wait_idx(b):
            pltpu.make_async_copy(src_hbm.at[pl.ds(0, C)], b["src"],
                                  b["s_idx"]).wait()
            pltpu.make_async_copy(src_hbm.at[pl.ds(0, C)], b["dst"],
                                  b["s_idx"]).wait()

        def issue_g(b):
            pltpu.async_copy(xyp_hbm.at[b["src"]], b["g"][0], b["s_g"])
            pltpu.async_copy(zbp_hbm.at[b["src"]], b["g"][1], b["s_g"])
            pltpu.async_copy(xyp_hbm.at[b["dst"]], b["g"][2], b["s_g"])
            pltpu.async_copy(zbp_hbm.at[b["dst"]], b["g"][3], b["s_g"])

        def wait_g(b):
            for t in range(4):
                pltpu.make_async_copy(src_hbm.at[pl.ds(0, C)], b["g"][t],
                                      b["s_g"]).wait()
            for t in range(3):
                pltpu.make_async_copy(s0_hbm.at[pl.ds(0, C)], b["sh"][t],
                                      b["s_sh"]).wait()

        def wait_out(b):
            pltpu.make_async_copy(b["out"], out_hbm.at[pl.ds(0, C)],
                                  b["s_out"]).wait()

        issue_idx(0, B[0])
        issue_idx(1, B[1])
        pltpu.async_copy(s0_hbm.at[pl.ds(0, C)], B[0]["out"], B[0]["s_out"])
        pltpu.async_copy(s0_hbm.at[pl.ds(0, C)], B[1]["out"], B[1]["s_out"])
        wait_idx(B[0])
        issue_g(B[0])

        def compute(b, i):
            xys_v, zbs_v, xyd_v, zbd_v = b["g"]
            out_v = b["out"]
            h0_v, h1_v, h2_v = b["sh"]

            def grp(g, carry2):
                q = pl.ds(g * _L, _L)
                ws = xys_v[q]
                wzs = zbs_v[q]
                wd = xyd_v[q]
                wzd = zbd_v[q]
                b9 = (wzs >> 16) * 9
                lat = [plsc.load_gather(lat_v, [b9 + kk]) for kk in range(9)]
                sh0 = h0_v[q]
                sh1 = h1_v[q]
                sh2 = h2_v[q]
                vx = _lo(wd) - _lo(ws) + sh0 * lat[0] + sh1 * lat[3] \
                    + sh2 * lat[6]
                vy = _hi(wd) - _hi(ws) + sh0 * lat[1] + sh1 * lat[4] \
                    + sh2 * lat[7]
                vz = _lo(wzd) - _lo(wzs) + sh0 * lat[2] + sh1 * lat[5] \
                    + sh2 * lat[8]
                n2 = jnp.maximum(vx * vx + vy * vy + vz * vz, 1e-30)
                out_v[q] = n2 * _rsqrt(n2)
                return carry2

            lax.fori_loop(0, R, grp, 0)
            e0 = wid * EW + i * C
            pltpu.async_copy(out_v, out_hbm.at[pl.ds(e0, C)], b["s_out"])

        def step(t, carry):
            for p in range(2):
                i = 2 * t + p
                bp, bq = B[p], B[1 - p]
                wait_idx(bq)
                issue_g(bq)
                wait_g(bp)
                wait_out(bp)
                compute(bp, i)
                issue_idx(i + 2, bp)
            return carry

        lax.fori_loop(0, NCH // 2 - 1, step, 0)
        wait_idx(B[1])
        issue_g(B[1])
        wait_g(B[0])
        wait_out(B[0])
        compute(B[0], NCH - 2)
        wait_g(B[1])
        wait_out(B[1])
        compute(B[1], NCH - 1)
        wait_out(B[0])
        wait_out(B[1])

    return k


def _pack_u16(lo_bf16_bits, hi_u32):
    return lax.bitcast_convert_type(
        lo_bf16_bits.astype(jnp.uint32) | (hi_u32 << 16), jnp.int32)


def kernel(pos, edge_index, edge_shift, lattice, batch):
    N = pos.shape[0]
    E = edge_index.shape[1]
    G = lattice.shape[0]
    xb, yb, zb = (lax.bitcast_convert_type(
        pos[:, t].astype(jnp.bfloat16), jnp.uint16) for t in range(3))
    xyp = _pack_u16(xb, yb.astype(jnp.uint32))
    zbp = _pack_u16(zb, batch.astype(jnp.uint32))
    latf = lattice.reshape(G * 9)
    ew = E // _NW
    c = min(4000, ew)
    c -= c % _L
    while c > _L and (ew % c or (ew // c) % 2):
        c -= _L
    k = _make_sc_kernel(N, E, G, c)
    return k(xyp, zbp, latf,
             edge_index[0], edge_index[1],
             edge_shift[:, 0], edge_shift[:, 1], edge_shift[:, 2])
